# speedup vs baseline: 1.5422x; 1.5422x over previous
import jax
import jax.numpy as jnp
from jax.experimental import pallas as pl

N = 10000


def _gat_layer_nomax(x, src, dst, W, a_s, a_d, b, n):
    h = x @ W
    als = h @ a_s
    ald = h @ a_d
    e = als[src] + ald[dst]
    e = jnp.where(e > 0, e, 0.2 * e)
    w = jnp.exp(e)
    denom = jax.ops.segment_sum(w, dst, num_segments=n)
    acc = jax.ops.segment_sum(w[:, None] * h[src], dst, num_segments=n)
    return acc / (denom[:, None] + 1e-16) + b


def _softmax_kernel(x_ref, o_ref):
    x = x_ref[...]
    m = jnp.max(x, axis=1, keepdims=True)
    ex = jnp.exp(x - m)
    o_ref[...] = ex / jnp.sum(ex, axis=1, keepdims=True)


def kernel(x, edge_idx, W1, a_s1, a_d1, b1, W2, a_s2, a_d2, b2, W3, a_s3, a_d3, b3):
    src = edge_idx[0]
    dst = edge_idx[1]
    h = jax.nn.relu(_gat_layer_nomax(x, src, dst, W1, a_s1, a_d1, b1, N))
    h = jax.nn.relu(_gat_layer_nomax(h, src, dst, W2, a_s2, a_d2, b2, N))
    h = _gat_layer_nomax(h, src, dst, W3, a_s3, a_d3, b3, N)
    return pl.pallas_call(
        _softmax_kernel,
        out_shape=jax.ShapeDtypeStruct((N, 40), jnp.float32),
    )(h)


# speedup vs baseline: 7.8246x; 5.0737x over previous
import dataclasses
import functools

import jax
import jax.numpy as jnp
from jax import lax
from jax.experimental import pallas as pl
from jax.experimental.pallas import tpu as pltpu
from jax.experimental.pallas import tpu_sc as plsc

N = 10000
NP = 10240
E = 320000
EP = 327680
NW = 32
EW = EP // NW
B = 128
NB = EW // B
G = 8
RSUB = NP // 16
F32 = jnp.float32
I32 = jnp.int32
_HI = lax.Precision.HIGHEST
_RB = 1024


def _sc_compiler_params():
    cp = pltpu.CompilerParams()
    if "needs_layout_passes" in pltpu.CompilerParams.__dataclass_fields__:
        cp = dataclasses.replace(cp, needs_layout_passes=False)
    return cp


def _tc1_body(x_ref, w_ref, as_ref, ad_ref, h_ref, als_ref, ald_ref):
    h = jnp.dot(x_ref[...], w_ref[...], preferred_element_type=F32,
                precision=_HI)
    h_ref[...] = h
    als_ref[...] = jnp.dot(h, as_ref[...], preferred_element_type=F32,
                           precision=_HI)
    ald_ref[...] = jnp.dot(h, ad_ref[...], preferred_element_type=F32,
                           precision=_HI)


def _tc1(x, W, a_s, a_d):
    return pl.pallas_call(
        _tc1_body,
        grid=(NP // _RB,),
        in_specs=[
            pl.BlockSpec((_RB, 128), lambda i: (i, 0)),
            pl.BlockSpec((128, 512), lambda i: (0, 0)),
            pl.BlockSpec((512, 1), lambda i: (0, 0)),
            pl.BlockSpec((512, 1), lambda i: (0, 0)),
        ],
        out_specs=[
            pl.BlockSpec((_RB, 512), lambda i: (i, 0)),
            pl.BlockSpec((_RB, 1), lambda i: (i, 0)),
            pl.BlockSpec((_RB, 1), lambda i: (i, 0)),
        ],
        out_shape=[
            jax.ShapeDtypeStruct((NP, 512), F32),
            jax.ShapeDtypeStruct((NP, 1), F32),
            jax.ShapeDtypeStruct((NP, 1), F32),
        ],
    )(x, W, a_s, a_d)


def _tcmid_body(cp, d, acc_ref, den_ref, b_ref, w_ref, as_ref, ad_ref,
                h_ref, als_ref, ald_ref):
    den = den_ref[0, :, 0:1] + den_ref[1, :, 0:1]
    inv = 1.0 / (den + 1e-16)
    hacc = None
    for c in range(cp):
        hc = (acc_ref[0, c] + acc_ref[1, c]) * inv + b_ref[:, c * d:(c + 1) * d]
        hc = jnp.maximum(hc, 0.0)
        p = jnp.dot(hc, w_ref[c * d:(c + 1) * d, :],
                    preferred_element_type=F32, precision=_HI)
        hacc = p if hacc is None else hacc + p
    h_ref[...] = hacc
    als_ref[...] = jnp.dot(hacc, as_ref[...], preferred_element_type=F32,
                           precision=_HI)
    ald_ref[...] = jnp.dot(hacc, ad_ref[...], preferred_element_type=F32,
                           precision=_HI)


def _tc_mid(cp, d, fout):
    fprev = cp * d
    return pl.pallas_call(
        functools.partial(_tcmid_body, cp, d),
        grid=(NP // _RB,),
        in_specs=[
            pl.BlockSpec((2, cp, _RB, d), lambda i: (0, 0, i, 0)),
            pl.BlockSpec((2, _RB, 16), lambda i: (0, i, 0)),
            pl.BlockSpec((1, fprev), lambda i: (0, 0)),
            pl.BlockSpec((fprev, fout), lambda i: (0, 0)),
            pl.BlockSpec((fout, 1), lambda i: (0, 0)),
            pl.BlockSpec((fout, 1), lambda i: (0, 0)),
        ],
        out_specs=[
            pl.BlockSpec((_RB, fout), lambda i: (i, 0)),
            pl.BlockSpec((_RB, 1), lambda i: (i, 0)),
            pl.BlockSpec((_RB, 1), lambda i: (i, 0)),
        ],
        out_shape=[
            jax.ShapeDtypeStruct((NP, fout), F32),
            jax.ShapeDtypeStruct((NP, 1), F32),
            jax.ShapeDtypeStruct((NP, 1), F32),
        ],
    )


def _tcfin_body(acc_ref, den_ref, b_ref, o_ref):
    den = den_ref[0, :, 0:1] + den_ref[1, :, 0:1]
    inv = 1.0 / (den + 1e-16)
    z = (acc_ref[0, 0] + acc_ref[1, 0]) * inv + b_ref[:, :]
    valid = lax.broadcasted_iota(I32, z.shape, 1) < 40
    zm = jnp.where(valid, z, -jnp.inf)
    m = jnp.max(zm, axis=1, keepdims=True)
    ex = jnp.where(valid, jnp.exp(z - m), 0.0)
    s = jnp.sum(ex, axis=1, keepdims=True)
    o_ref[...] = (ex / s)[:, :40]


def _tc_fin(acc, den, b):
    return pl.pallas_call(
        _tcfin_body,
        grid=(NP // _RB,),
        in_specs=[
            pl.BlockSpec((2, 1, _RB, 128), lambda i: (0, 0, i, 0)),
            pl.BlockSpec((2, _RB, 16), lambda i: (0, i, 0)),
            pl.BlockSpec((1, 128), lambda i: (0, 0)),
        ],
        out_specs=pl.BlockSpec((_RB, 40), lambda i: (i, 0)),
        out_shape=jax.ShapeDtypeStruct((NP, 40), F32),
    )(acc, den, b)


def _sc_w():
    mesh = plsc.VectorSubcoreMesh(core_axis_name="c", subcore_axis_name="s")

    @functools.partial(
        pl.kernel,
        mesh=mesh,
        compiler_params=_sc_compiler_params(),
        out_type=[
            jax.ShapeDtypeStruct((EP,), F32),
            jax.ShapeDtypeStruct((2 * NP, 16), F32),
        ],
        scratch_types=[
            pltpu.VMEM((EW,), I32),
            pltpu.VMEM((EW,), I32),
            pltpu.VMEM((NB, B), I32),
            pltpu.VMEM((EW,), F32),
            pltpu.VMEM((NP,), F32),
            pltpu.VMEM((NP,), F32),
            pltpu.VMEM((B, 16), F32),
            pltpu.VMEM_SHARED((NP, 16), F32),
        ],
    )
    def sck(srcf_hbm, dstf_hbm, dst2_hbm, als_hbm, ald_hbm, w_hbm, den_hbm,
            src_v, dst1_v, dst2_v, w_v, als_v, ald_v, den_buf, den_s):
        cid = lax.axis_index("c")
        sid = lax.axis_index("s")
        wid = sid * 2 + cid
        base = sid * RSUB

        pltpu.sync_copy(srcf_hbm.at[pl.ds(wid * EW, EW)], src_v)
        pltpu.sync_copy(dstf_hbm.at[pl.ds(wid * EW, EW)], dst1_v)
        pltpu.sync_copy(dst2_hbm.at[pl.ds(wid * NB, NB)], dst2_v)
        pltpu.sync_copy(als_hbm, als_v)
        pltpu.sync_copy(ald_hbm, ald_v)

        z16 = jnp.zeros((16,), F32)

        @pl.loop(0, B)
        def _(j):
            den_buf[j, :] = z16

        for r in range(RSUB // B):
            pltpu.sync_copy(den_buf, den_s.at[pl.ds(base + r * B, B)])
        plsc.subcore_barrier()

        @pl.loop(0, EW // 16)
        def _(g):
            sl = pl.ds(g * 16, 16)
            a1 = plsc.load_gather(als_v, [src_v[sl]])
            a2 = plsc.load_gather(ald_v, [dst1_v[sl]])
            e = a1 + a2
            e = jnp.where(e > 0.0, e, 0.2 * e)
            w_v[sl] = jnp.exp(e)

        pltpu.sync_copy(w_v, w_hbm.at[pl.ds(wid * EW, EW)])

        @pl.loop(0, NB)
        def _(b):
            @pl.loop(0, B)
            def _(j):
                wj = plsc.load_gather(w_v, [jnp.full((16,), b * B + j, I32)])
                den_buf[j, :] = wj

            pltpu.sync_copy(den_buf, den_s.at[dst2_v.at[b]], add=True)

        plsc.subcore_barrier()
        pltpu.sync_copy(den_s.at[pl.ds(base, RSUB)],
                        den_hbm.at[pl.ds(cid * NP + base, RSUB)])

    return sck


def _sc_agg(C, D):
    KD = D // 16
    NT = NB // G
    CP = C + 1
    mesh = plsc.VectorSubcoreMesh(core_axis_name="c", subcore_axis_name="s")

    @functools.partial(
        pl.kernel,
        mesh=mesh,
        compiler_params=_sc_compiler_params(),
        out_type=jax.ShapeDtypeStruct((2 * CP * NP, D), F32),
        scratch_types=[
            pltpu.VMEM((B, D), F32),
            pltpu.VMEM((B, D), F32),
            pltpu.VMEM((G * B,), I32),
            pltpu.VMEM((G, B), I32),
            pltpu.VMEM((G * B,), F32),
            pltpu.VMEM_SHARED((NP, D), F32),
            pltpu.SemaphoreType.DMA,
            pltpu.SemaphoreType.DMA,
        ],
    )
    def sck(h_hbm, src_hbm, dst2_hbm, w_hbm, acc_hbm,
            rows0, rows1, sidx, dbuf, wbuf, acc_s, sem0, sem1):
        cid = lax.axis_index("c")
        sid = lax.axis_index("s")
        wid = sid * 2 + cid
        base = sid * RSUB
        rows = (rows0, rows1)
        sems = (sem0, sem1)
        z16 = jnp.zeros((16,), F32)

        def zero_acc():
            @pl.loop(0, B)
            def _(j):
                for k in range(KD):
                    rows0[j, pl.ds(k * 16, 16)] = z16

            for r in range(RSUB // B):
                pltpu.sync_copy(rows0, acc_s.at[pl.ds(base + r * B, B)])
            plsc.subcore_barrier()

        def flush_acc(cc):
            plsc.subcore_barrier()
            off = (cid * CP + cc) * NP + base
            pltpu.sync_copy(acc_s.at[pl.ds(base, RSUB)],
                            acc_hbm.at[pl.ds(off, RSUB)])

        def process_block(q, rbuf):
            @pl.loop(0, B)
            def _(j):
                wj = plsc.load_gather(wbuf, [jnp.full((16,), q * B + j, I32)])
                for k in range(KD):
                    sl = (j, pl.ds(k * 16, 16))
                    rbuf[sl] = rbuf[sl] * wj

            pltpu.sync_copy(rbuf, acc_s.at[dbuf.at[q]], add=True)

        for c in range(C):
            zero_acc()

            @pl.loop(0, NT)
            def _(t):
                erow = wid * NB + t * G
                ebase = erow * B
                pltpu.sync_copy(src_hbm.at[pl.ds(ebase, G * B)], sidx)
                pltpu.sync_copy(w_hbm.at[pl.ds(ebase, G * B)], wbuf)
                pltpu.sync_copy(dst2_hbm.at[pl.ds(erow, G)], dbuf)

                @pl.loop(0, G * B // 16)
                def _(g):
                    sl = pl.ds(g * 16, 16)
                    sidx[sl] = sidx[sl] * C + c

                cur = pltpu.async_copy(
                    h_hbm.at[sidx.at[pl.ds(0, B)]], rows[0], sems[0])
                for q in range(G):
                    nxt = None
                    if q + 1 < G:
                        nxt = pltpu.async_copy(
                            h_hbm.at[sidx.at[pl.ds((q + 1) * B, B)]],
                            rows[(q + 1) % 2], sems[(q + 1) % 2])
                    cur.wait()
                    process_block(q, rows[q % 2])
                    cur = nxt

            flush_acc(c)

        zero_acc()

        @pl.loop(0, NT)
        def _(t):
            erow = wid * NB + t * G
            pltpu.sync_copy(w_hbm.at[pl.ds(erow * B, G * B)], wbuf)
            pltpu.sync_copy(dst2_hbm.at[pl.ds(erow, G)], dbuf)

            for q in range(G):
                @pl.loop(0, B)
                def _(j):
                    wj = plsc.load_gather(
                        wbuf, [jnp.full((16,), q * B + j, I32)])
                    rows0[j, pl.ds(0, 16)] = wj

                pltpu.sync_copy(rows0, acc_s.at[dbuf.at[q]], add=True)

        flush_acc(C)

    return sck


_SC_AGG1 = _sc_agg(4, 128)
_SC_AGG2 = _sc_agg(2, 128)
_SC_AGG3 = _sc_agg(1, 128)
_SC_AGGD = _sc_agg(0, 128)


def _sc_w_min():
    mesh = plsc.VectorSubcoreMesh(core_axis_name="c", subcore_axis_name="s")

    @functools.partial(
        pl.kernel,
        mesh=mesh,
        compiler_params=_sc_compiler_params(),
        out_type=jax.ShapeDtypeStruct((EP,), F32),
        scratch_types=[
            pltpu.VMEM((EW,), I32),
            pltpu.VMEM((EW,), I32),
            pltpu.VMEM((EW,), F32),
            pltpu.VMEM((NP,), F32),
            pltpu.VMEM((NP,), F32),
        ],
    )
    def sck(srcf_hbm, dstf_hbm, als_hbm, ald_hbm, w_hbm,
            src_v, dst1_v, w_v, als_v, ald_v):
        cid = lax.axis_index("c")
        sid = lax.axis_index("s")
        wid = sid * 2 + cid

        pltpu.sync_copy(srcf_hbm.at[pl.ds(wid * EW, EW)], src_v)
        pltpu.sync_copy(dstf_hbm.at[pl.ds(wid * EW, EW)], dst1_v)
        pltpu.sync_copy(als_hbm, als_v)
        pltpu.sync_copy(ald_hbm, ald_v)

        @pl.loop(0, EW // 16)
        def _(g):
            sl = pl.ds(g * 16, 16)
            a1 = plsc.load_gather(als_v, [src_v[sl]])
            a2 = plsc.load_gather(ald_v, [dst1_v[sl]])
            e = a1 + a2
            e = jnp.where(e > 0.0, e, 0.2 * e)
            w_v[sl] = jnp.exp(e)

        pltpu.sync_copy(w_v, w_hbm.at[pl.ds(wid * EW, EW)])

    return sck


_SC_WMIN = _sc_w_min()


def _jnp_agg(h, src, dst, w, C):
    wE = w[:E]
    acc = jax.ops.segment_sum(wE[:, None] * h[src], dst, num_segments=NP)
    accp = acc.reshape(NP, C, 128).transpose(1, 0, 2)
    accp = jnp.concatenate([accp, jnp.zeros_like(accp)], axis=0)
    return accp.reshape(2 * C * NP, 128)


def _jnp_den(src, dst, als, ald):
    e = als[src] + ald[dst]
    e = jnp.where(e > 0, e, 0.2 * e)
    w = jnp.exp(e)
    den = jax.ops.segment_sum(w, dst, num_segments=NP)
    denp = jnp.broadcast_to(den[:, None], (NP, 16))
    denp = jnp.concatenate([denp, jnp.zeros_like(denp)], axis=0)
    wp = jnp.concatenate([w, jnp.ones((EP - E,), F32)])
    return wp, denp


def kernel(x, edge_idx, W1, a_s1, a_d1, b1, W2, a_s2, a_d2, b2,
           W3, a_s3, a_d3, b3):
    src = edge_idx[0]
    dst = edge_idx[1]
    pad = jnp.full((EP - E,), N, I32)
    srcp = jnp.concatenate([src, pad])
    dstp = jnp.concatenate([dst, pad])
    dst2d = dstp.reshape(NW * NB, B)
    xp = jnp.pad(x, ((0, NP - N), (0, 0)))

    h1, als1, ald1 = _tc1(xp, W1, a_s1.reshape(-1, 1), a_d1.reshape(-1, 1))
    w1 = _SC_WMIN(srcp, dstp, als1.reshape(-1), ald1.reshape(-1))
    a1 = _SC_AGG1(h1.reshape(4 * NP, 128), srcp, dst2d,
                  w1).reshape(2, 5, NP, 128)
    acc1, den1 = a1[:, :4], a1[:, 4, :, :16]

    h2, als2, ald2 = _tc_mid(4, 128, 256)(
        acc1, den1,
        b1.reshape(1, -1), W2, a_s2.reshape(-1, 1), a_d2.reshape(-1, 1))
    w2 = _SC_WMIN(srcp, dstp, als2.reshape(-1), ald2.reshape(-1))
    a2 = _SC_AGG2(h2.reshape(2 * NP, 128), srcp, dst2d,
                  w2).reshape(2, 3, NP, 128)
    acc2, den2 = a2[:, :2], a2[:, 2, :, :16]

    W3p = jnp.pad(W3, ((0, 0), (0, 88)))
    h3, als3, ald3 = _tc_mid(2, 128, 128)(
        acc2, den2,
        b2.reshape(1, -1), W3p, jnp.pad(a_s3, (0, 88)).reshape(-1, 1),
        jnp.pad(a_d3, (0, 88)).reshape(-1, 1))
    w3 = _SC_WMIN(srcp, dstp, als3.reshape(-1), ald3.reshape(-1))
    a3 = _SC_AGG3(h3, srcp, dst2d, w3).reshape(2, 2, NP, 128)
    acc3, den3 = a3[:, :1], a3[:, 1, :, :16]

    out = _tc_fin(acc3, den3, jnp.pad(b3, (0, 88)).reshape(1, -1))
    return out[:N]


# speedup vs baseline: 7.8911x; 1.0085x over previous
import dataclasses
import functools

import jax
import jax.numpy as jnp
from jax import lax
from jax.experimental import pallas as pl
from jax.experimental.pallas import tpu as pltpu
from jax.experimental.pallas import tpu_sc as plsc

N = 10000
NP = 10240
E = 320000
EP = 327680
NW = 32
EW = EP // NW
B = 128
NB = EW // B
G = 8
RSUB = NP // 16
F32 = jnp.float32
I32 = jnp.int32
_HI = lax.Precision.HIGHEST
_RB = 1024


def _sc_compiler_params():
    cp = pltpu.CompilerParams()
    if "needs_layout_passes" in pltpu.CompilerParams.__dataclass_fields__:
        cp = dataclasses.replace(cp, needs_layout_passes=False)
    return cp


def _tc1_body(x_ref, w_ref, as_ref, ad_ref, h_ref, als_ref, ald_ref):
    h = jnp.dot(x_ref[...], w_ref[...], preferred_element_type=F32,
                precision=_HI)
    h_ref[...] = h
    als_ref[...] = jnp.dot(h, as_ref[...], preferred_element_type=F32,
                           precision=_HI)
    ald_ref[...] = jnp.dot(h, ad_ref[...], preferred_element_type=F32,
                           precision=_HI)


def _tc1(x, W, a_s, a_d):
    return pl.pallas_call(
        _tc1_body,
        grid=(NP // _RB,),
        in_specs=[
            pl.BlockSpec((_RB, 128), lambda i: (i, 0)),
            pl.BlockSpec((128, 512), lambda i: (0, 0)),
            pl.BlockSpec((512, 1), lambda i: (0, 0)),
            pl.BlockSpec((512, 1), lambda i: (0, 0)),
        ],
        out_specs=[
            pl.BlockSpec((_RB, 512), lambda i: (i, 0)),
            pl.BlockSpec((_RB, 1), lambda i: (i, 0)),
            pl.BlockSpec((_RB, 1), lambda i: (i, 0)),
        ],
        out_shape=[
            jax.ShapeDtypeStruct((NP, 512), F32),
            jax.ShapeDtypeStruct((NP, 1), F32),
            jax.ShapeDtypeStruct((NP, 1), F32),
        ],
    )(x, W, a_s, a_d)


def _tcmid_body(cp, d, acc_ref, den_ref, b_ref, w_ref, as_ref, ad_ref,
                h_ref, als_ref, ald_ref):
    den = den_ref[0, :, 0:1] + den_ref[1, :, 0:1]
    inv = 1.0 / (den + 1e-16)
    hacc = None
    for c in range(cp):
        hc = (acc_ref[0, c] + acc_ref[1, c]) * inv + b_ref[:, c * d:(c + 1) * d]
        hc = jnp.maximum(hc, 0.0)
        p = jnp.dot(hc, w_ref[c * d:(c + 1) * d, :],
                    preferred_element_type=F32, precision=_HI)
        hacc = p if hacc is None else hacc + p
    h_ref[...] = hacc
    als_ref[...] = jnp.dot(hacc, as_ref[...], preferred_element_type=F32,
                           precision=_HI)
    ald_ref[...] = jnp.dot(hacc, ad_ref[...], preferred_element_type=F32,
                           precision=_HI)


def _tc_mid(cp, d, fout):
    fprev = cp * d
    return pl.pallas_call(
        functools.partial(_tcmid_body, cp, d),
        grid=(NP // _RB,),
        in_specs=[
            pl.BlockSpec((2, cp, _RB, d), lambda i: (0, 0, i, 0)),
            pl.BlockSpec((2, _RB, 16), lambda i: (0, i, 0)),
            pl.BlockSpec((1, fprev), lambda i: (0, 0)),
            pl.BlockSpec((fprev, fout), lambda i: (0, 0)),
            pl.BlockSpec((fout, 1), lambda i: (0, 0)),
            pl.BlockSpec((fout, 1), lambda i: (0, 0)),
        ],
        out_specs=[
            pl.BlockSpec((_RB, fout), lambda i: (i, 0)),
            pl.BlockSpec((_RB, 1), lambda i: (i, 0)),
            pl.BlockSpec((_RB, 1), lambda i: (i, 0)),
        ],
        out_shape=[
            jax.ShapeDtypeStruct((NP, fout), F32),
            jax.ShapeDtypeStruct((NP, 1), F32),
            jax.ShapeDtypeStruct((NP, 1), F32),
        ],
    )


def _tcfin_body(acc_ref, den_ref, b_ref, o_ref):
    den = den_ref[0, :, 0:1] + den_ref[1, :, 0:1]
    inv = 1.0 / (den + 1e-16)
    z = (acc_ref[0, 0] + acc_ref[1, 0]) * inv + b_ref[:, :]
    valid = lax.broadcasted_iota(I32, z.shape, 1) < 40
    zm = jnp.where(valid, z, -jnp.inf)
    m = jnp.max(zm, axis=1, keepdims=True)
    ex = jnp.where(valid, jnp.exp(z - m), 0.0)
    s = jnp.sum(ex, axis=1, keepdims=True)
    o_ref[...] = (ex / s)[:, :40]


def _tc_fin(acc, den, b):
    return pl.pallas_call(
        _tcfin_body,
        grid=(NP // _RB,),
        in_specs=[
            pl.BlockSpec((2, 1, _RB, 128), lambda i: (0, 0, i, 0)),
            pl.BlockSpec((2, _RB, 16), lambda i: (0, i, 0)),
            pl.BlockSpec((1, 128), lambda i: (0, 0)),
        ],
        out_specs=pl.BlockSpec((_RB, 40), lambda i: (i, 0)),
        out_shape=jax.ShapeDtypeStruct((NP, 40), F32),
    )(acc, den, b)


def _sc_w():
    mesh = plsc.VectorSubcoreMesh(core_axis_name="c", subcore_axis_name="s")

    @functools.partial(
        pl.kernel,
        mesh=mesh,
        compiler_params=_sc_compiler_params(),
        out_type=[
            jax.ShapeDtypeStruct((EP,), F32),
            jax.ShapeDtypeStruct((2 * NP, 16), F32),
        ],
        scratch_types=[
            pltpu.VMEM((EW,), I32),
            pltpu.VMEM((EW,), I32),
            pltpu.VMEM((NB, B), I32),
            pltpu.VMEM((EW,), F32),
            pltpu.VMEM((NP,), F32),
            pltpu.VMEM((NP,), F32),
            pltpu.VMEM((B, 16), F32),
            pltpu.VMEM_SHARED((NP, 16), F32),
        ],
    )
    def sck(srcf_hbm, dstf_hbm, dst2_hbm, als_hbm, ald_hbm, w_hbm, den_hbm,
            src_v, dst1_v, dst2_v, w_v, als_v, ald_v, den_buf, den_s):
        cid = lax.axis_index("c")
        sid = lax.axis_index("s")
        wid = sid * 2 + cid
        base = sid * RSUB

        pltpu.sync_copy(srcf_hbm.at[pl.ds(wid * EW, EW)], src_v)
        pltpu.sync_copy(dstf_hbm.at[pl.ds(wid * EW, EW)], dst1_v)
        pltpu.sync_copy(dst2_hbm.at[pl.ds(wid * NB, NB)], dst2_v)
        pltpu.sync_copy(als_hbm, als_v)
        pltpu.sync_copy(ald_hbm, ald_v)

        z16 = jnp.zeros((16,), F32)

        @pl.loop(0, B)
        def _(j):
            den_buf[j, :] = z16

        for r in range(RSUB // B):
            pltpu.sync_copy(den_buf, den_s.at[pl.ds(base + r * B, B)])
        plsc.subcore_barrier()

        @pl.loop(0, EW // 16, unroll=4)
        def _(g):
            sl = pl.ds(g * 16, 16)
            a1 = plsc.load_gather(als_v, [src_v[sl]])
            a2 = plsc.load_gather(ald_v, [dst1_v[sl]])
            e = a1 + a2
            e = jnp.where(e > 0.0, e, 0.2 * e)
            w_v[sl] = jnp.exp(e)

        pltpu.sync_copy(w_v, w_hbm.at[pl.ds(wid * EW, EW)])

        @pl.loop(0, NB)
        def _(b):
            @pl.loop(0, B)
            def _(j):
                wj = plsc.load_gather(w_v, [jnp.full((16,), b * B + j, I32)])
                den_buf[j, :] = wj

            pltpu.sync_copy(den_buf, den_s.at[dst2_v.at[b]], add=True)

        plsc.subcore_barrier()
        pltpu.sync_copy(den_s.at[pl.ds(base, RSUB)],
                        den_hbm.at[pl.ds(cid * NP + base, RSUB)])

    return sck


def _sc_agg(C, D):
    KD = D // 16
    NT = NB // G
    CP = C + 1
    mesh = plsc.VectorSubcoreMesh(core_axis_name="c", subcore_axis_name="s")

    @functools.partial(
        pl.kernel,
        mesh=mesh,
        compiler_params=_sc_compiler_params(),
        out_type=jax.ShapeDtypeStruct((2 * CP * NP, D), F32),
        scratch_types=[
            pltpu.VMEM((B, D), F32),
            pltpu.VMEM((B, D), F32),
            pltpu.VMEM((G * B,), I32),
            pltpu.VMEM((G, B), I32),
            pltpu.VMEM((G * B,), F32),
            pltpu.VMEM_SHARED((NP, D), F32),
            pltpu.SemaphoreType.DMA,
            pltpu.SemaphoreType.DMA,
        ],
    )
    def sck(h_hbm, src_hbm, dst2_hbm, w_hbm, acc_hbm,
            rows0, rows1, sidx, dbuf, wbuf, acc_s, sem0, sem1):
        cid = lax.axis_index("c")
        sid = lax.axis_index("s")
        wid = sid * 2 + cid
        base = sid * RSUB
        rows = (rows0, rows1)
        sems = (sem0, sem1)
        z16 = jnp.zeros((16,), F32)

        def zero_acc():
            @pl.loop(0, B)
            def _(j):
                for k in range(KD):
                    rows0[j, pl.ds(k * 16, 16)] = z16

            for r in range(RSUB // B):
                pltpu.sync_copy(rows0, acc_s.at[pl.ds(base + r * B, B)])
            plsc.subcore_barrier()

        def flush_acc(cc):
            plsc.subcore_barrier()
            off = (cid * CP + cc) * NP + base
            pltpu.sync_copy(acc_s.at[pl.ds(base, RSUB)],
                            acc_hbm.at[pl.ds(off, RSUB)])

        def process_block(q, rbuf):
            @pl.loop(0, B, unroll=4)
            def _(j):
                wj = plsc.load_gather(wbuf, [jnp.full((16,), q * B + j, I32)])
                for k in range(KD):
                    sl = (j, pl.ds(k * 16, 16))
                    rbuf[sl] = rbuf[sl] * wj

            pltpu.sync_copy(rbuf, acc_s.at[dbuf.at[q]], add=True)

        for c in range(C):
            zero_acc()

            @pl.loop(0, NT)
            def _(t):
                erow = wid * NB + t * G
                ebase = erow * B
                pltpu.sync_copy(src_hbm.at[pl.ds(ebase, G * B)], sidx)
                pltpu.sync_copy(w_hbm.at[pl.ds(ebase, G * B)], wbuf)
                pltpu.sync_copy(dst2_hbm.at[pl.ds(erow, G)], dbuf)

                @pl.loop(0, G * B // 16, unroll=4)
                def _(g):
                    sl = pl.ds(g * 16, 16)
                    sidx[sl] = sidx[sl] * C + c

                cur = pltpu.async_copy(
                    h_hbm.at[sidx.at[pl.ds(0, B)]], rows[0], sems[0])
                for q in range(G):
                    nxt = None
                    if q + 1 < G:
                        nxt = pltpu.async_copy(
                            h_hbm.at[sidx.at[pl.ds((q + 1) * B, B)]],
                            rows[(q + 1) % 2], sems[(q + 1) % 2])
                    cur.wait()
                    process_block(q, rows[q % 2])
                    cur = nxt

            flush_acc(c)

        zero_acc()

        @pl.loop(0, NT)
        def _(t):
            erow = wid * NB + t * G
            pltpu.sync_copy(w_hbm.at[pl.ds(erow * B, G * B)], wbuf)
            pltpu.sync_copy(dst2_hbm.at[pl.ds(erow, G)], dbuf)

            for q in range(G):
                @pl.loop(0, B, unroll=4)
                def _(j):
                    wj = plsc.load_gather(
                        wbuf, [jnp.full((16,), q * B + j, I32)])
                    rows0[j, pl.ds(0, 16)] = wj

                pltpu.sync_copy(rows0, acc_s.at[dbuf.at[q]], add=True)

        flush_acc(C)

    return sck


_SC_AGG1 = _sc_agg(4, 128)
_SC_AGG2 = _sc_agg(2, 128)
_SC_AGG3 = _sc_agg(1, 128)
_SC_AGGD = _sc_agg(0, 128)


def _sc_w_min():
    mesh = plsc.VectorSubcoreMesh(core_axis_name="c", subcore_axis_name="s")

    @functools.partial(
        pl.kernel,
        mesh=mesh,
        compiler_params=_sc_compiler_params(),
        out_type=jax.ShapeDtypeStruct((EP,), F32),
        scratch_types=[
            pltpu.VMEM((EW,), I32),
            pltpu.VMEM((EW,), I32),
            pltpu.VMEM((EW,), F32),
            pltpu.VMEM((NP,), F32),
            pltpu.VMEM((NP,), F32),
        ],
    )
    def sck(srcf_hbm, dstf_hbm, als_hbm, ald_hbm, w_hbm,
            src_v, dst1_v, w_v, als_v, ald_v):
        cid = lax.axis_index("c")
        sid = lax.axis_index("s")
        wid = sid * 2 + cid

        pltpu.sync_copy(srcf_hbm.at[pl.ds(wid * EW, EW)], src_v)
        pltpu.sync_copy(dstf_hbm.at[pl.ds(wid * EW, EW)], dst1_v)
        pltpu.sync_copy(als_hbm, als_v)
        pltpu.sync_copy(ald_hbm, ald_v)

        @pl.loop(0, EW // 16, unroll=4)
        def _(g):
            sl = pl.ds(g * 16, 16)
            a1 = plsc.load_gather(als_v, [src_v[sl]])
            a2 = plsc.load_gather(ald_v, [dst1_v[sl]])
            e = a1 + a2
            e = jnp.where(e > 0.0, e, 0.2 * e)
            w_v[sl] = jnp.exp(e)

        pltpu.sync_copy(w_v, w_hbm.at[pl.ds(wid * EW, EW)])

    return sck


_SC_WMIN = _sc_w_min()


def _jnp_agg(h, src, dst, w, C):
    wE = w[:E]
    acc = jax.ops.segment_sum(wE[:, None] * h[src], dst, num_segments=NP)
    accp = acc.reshape(NP, C, 128).transpose(1, 0, 2)
    accp = jnp.concatenate([accp, jnp.zeros_like(accp)], axis=0)
    return accp.reshape(2 * C * NP, 128)


def _jnp_den(src, dst, als, ald):
    e = als[src] + ald[dst]
    e = jnp.where(e > 0, e, 0.2 * e)
    w = jnp.exp(e)
    den = jax.ops.segment_sum(w, dst, num_segments=NP)
    denp = jnp.broadcast_to(den[:, None], (NP, 16))
    denp = jnp.concatenate([denp, jnp.zeros_like(denp)], axis=0)
    wp = jnp.concatenate([w, jnp.ones((EP - E,), F32)])
    return wp, denp


def kernel(x, edge_idx, W1, a_s1, a_d1, b1, W2, a_s2, a_d2, b2,
           W3, a_s3, a_d3, b3):
    src = edge_idx[0]
    dst = edge_idx[1]
    pad = jnp.full((EP - E,), N, I32)
    srcp = jnp.concatenate([src, pad])
    dstp = jnp.concatenate([dst, pad])
    dst2d = dstp.reshape(NW * NB, B)
    xp = jnp.pad(x, ((0, NP - N), (0, 0)))

    h1, als1, ald1 = _tc1(xp, W1, a_s1.reshape(-1, 1), a_d1.reshape(-1, 1))
    w1 = _SC_WMIN(srcp, dstp, als1.reshape(-1), ald1.reshape(-1))
    a1 = _SC_AGG1(h1.reshape(4 * NP, 128), srcp, dst2d,
                  w1).reshape(2, 5, NP, 128)
    acc1, den1 = a1[:, :4], a1[:, 4, :, :16]

    h2, als2, ald2 = _tc_mid(4, 128, 256)(
        acc1, den1,
        b1.reshape(1, -1), W2, a_s2.reshape(-1, 1), a_d2.reshape(-1, 1))
    w2 = _SC_WMIN(srcp, dstp, als2.reshape(-1), ald2.reshape(-1))
    a2 = _SC_AGG2(h2.reshape(2 * NP, 128), srcp, dst2d,
                  w2).reshape(2, 3, NP, 128)
    acc2, den2 = a2[:, :2], a2[:, 2, :, :16]

    W3p = jnp.pad(W3, ((0, 0), (0, 88)))
    h3, als3, ald3 = _tc_mid(2, 128, 128)(
        acc2, den2,
        b2.reshape(1, -1), W3p, jnp.pad(a_s3, (0, 88)).reshape(-1, 1),
        jnp.pad(a_d3, (0, 88)).reshape(-1, 1))
    w3 = _SC_WMIN(srcp, dstp, als3.reshape(-1), ald3.reshape(-1))
    a3 = _SC_AGG3(h3, srcp, dst2d, w3).reshape(2, 2, NP, 128)
    acc3, den3 = a3[:, :1], a3[:, 1, :, :16]

    out = _tc_fin(acc3, den3, jnp.pad(b3, (0, 88)).reshape(1, -1))
    return out[:N]
